# initial kernel scaffold (unmeasured)
import jax
import jax.numpy as jnp
from jax import lax
from jax.experimental import pallas as pl
from jax.experimental.pallas import tpu as pltpu

N_DEV = 4
M_PER = 1024
N_COLS = 8192
W = 1024
N_TILES = N_COLS // W


def _rs_body(part_ref, out_ref, amax_ref,
             buf_ref, stage_ref, sbuf_ref, res_ref, axbuf_ref,
             local_sem, send_sems, recv_sems, ax_send_sems, ax_recv_sems):
    d = lax.axis_index("i")
    left = lax.rem(d - 1 + N_DEV, N_DEV)
    right = lax.rem(d + 1, N_DEV)

    barrier = pltpu.get_barrier_semaphore()
    for nbr in (left, right):
        pl.semaphore_signal(
            barrier, inc=1,
            device_id=(nbr,), device_id_type=pl.DeviceIdType.MESH,
        )
    pl.semaphore_wait(barrier, 2)

    amax_val = jnp.zeros((), jnp.float32)
    for t in range(N_TILES):
        col = pl.ds(t * W, W)

        c0 = lax.rem(d - 1 + N_DEV, N_DEV)
        cp = pltpu.make_async_copy(
            part_ref.at[pl.ds(c0 * M_PER, M_PER), col], stage_ref, local_sem)
        cp.start()
        cp.wait()
        rdma = pltpu.make_async_remote_copy(
            src_ref=stage_ref,
            dst_ref=buf_ref.at[0],
            send_sem=send_sems.at[0],
            recv_sem=recv_sems.at[0],
            device_id=(right,),
            device_id_type=pl.DeviceIdType.MESH,
        )
        rdma.start()
        rdma.wait()

        for s in (1, 2):
            c = lax.rem(d - 1 - s + N_DEV, N_DEV)
            cp = pltpu.make_async_copy(
                part_ref.at[pl.ds(c * M_PER, M_PER), col], stage_ref, local_sem)
            cp.start()
            cp.wait()
            sbuf_ref[...] = buf_ref[s - 1] + stage_ref[...]
            rdma = pltpu.make_async_remote_copy(
                src_ref=sbuf_ref,
                dst_ref=buf_ref.at[s],
                send_sem=send_sems.at[s],
                recv_sem=recv_sems.at[s],
                device_id=(right,),
                device_id_type=pl.DeviceIdType.MESH,
            )
            rdma.start()
            rdma.wait()

        cp = pltpu.make_async_copy(
            part_ref.at[pl.ds(d * M_PER, M_PER), col], stage_ref, local_sem)
        cp.start()
        cp.wait()
        res_ref[...] = buf_ref[2] + stage_ref[...]
        amax_val = jnp.maximum(amax_val, jnp.max(jnp.abs(res_ref[...])))
        cp = pltpu.make_async_copy(res_ref, out_ref.at[:, col], local_sem)
        cp.start()
        cp.wait()

    axbuf_ref[3, :, :] = jnp.full((8, 128), amax_val, jnp.float32)
    sends = []
    for off in (1, 2, 3):
        tgt = lax.rem(d + off, N_DEV)
        rdma = pltpu.make_async_remote_copy(
            src_ref=axbuf_ref.at[3],
            dst_ref=axbuf_ref.at[off - 1],
            send_sem=ax_send_sems.at[off - 1],
            recv_sem=ax_recv_sems.at[off - 1],
            device_id=(tgt,),
            device_id_type=pl.DeviceIdType.MESH,
        )
        rdma.start()
        sends.append(rdma)
    for rdma in sends:
        rdma.wait_send()
    for rdma in sends:
        rdma.wait_recv()
    amax_ref[0, 0] = jnp.max(axbuf_ref[...])


def _reduce_scatter_amax(partial):
    return pl.pallas_call(
        _rs_body,
        out_shape=[
            jax.ShapeDtypeStruct((M_PER, N_COLS), jnp.float32),
            jax.ShapeDtypeStruct((1, 1), jnp.float32),
        ],
        in_specs=[pl.BlockSpec(memory_space=pltpu.ANY)],
        out_specs=[
            pl.BlockSpec(memory_space=pltpu.ANY),
            pl.BlockSpec(memory_space=pltpu.VMEM),
        ],
        scratch_shapes=[
            pltpu.VMEM((3, M_PER, W), jnp.float32),
            pltpu.VMEM((M_PER, W), jnp.float32),
            pltpu.VMEM((M_PER, W), jnp.float32),
            pltpu.VMEM((M_PER, W), jnp.float32),
            pltpu.VMEM((N_DEV, 8, 128), jnp.float32),
            pltpu.SemaphoreType.DMA,
            pltpu.SemaphoreType.DMA((3,)),
            pltpu.SemaphoreType.DMA((3,)),
            pltpu.SemaphoreType.DMA((3,)),
            pltpu.SemaphoreType.DMA((3,)),
        ],
        compiler_params=pltpu.CompilerParams(collective_id=0),
    )(partial)


def kernel(x, w_mat):
    partial = jnp.dot(x, w_mat, precision=lax.Precision.HIGHEST)
    acc, amax = _reduce_scatter_amax(partial)
    scale = amax[0, 0] / 448.0
    q = (acc / scale).astype(jnp.float8_e4m3fn).astype(jnp.float32)
    return q * scale


# baseline (device time: 1695800 ns/iter reference)
import jax
import jax.numpy as jnp
from jax import lax
from jax.experimental import pallas as pl
from jax.experimental.pallas import tpu as pltpu

N_DEV = 4
M_PER = 1024
N_COLS = 8192
W = 1024
N_TILES = N_COLS // W


def _rs_body(part_ref, out_ref, amax_ref,
             buf_ref, stage_ref, sbuf_ref, res_ref, axbuf_ref,
             local_sem, send_sems, recv_sems, ax_send_sems, ax_recv_sems):
    d = lax.axis_index("i")
    left = lax.rem(d - 1 + N_DEV, N_DEV)
    right = lax.rem(d + 1, N_DEV)

    barrier = pltpu.get_barrier_semaphore()
    for nbr in (left, right):
        pl.semaphore_signal(
            barrier, inc=1,
            device_id=(nbr,), device_id_type=pl.DeviceIdType.MESH,
        )
    pl.semaphore_wait(barrier, 2)

    amax_val = jnp.zeros((), jnp.float32)
    for t in range(N_TILES):
        col = pl.ds(t * W, W)

        c0 = lax.rem(d - 1 + N_DEV, N_DEV)
        cp = pltpu.make_async_copy(
            part_ref.at[pl.ds(c0 * M_PER, M_PER), col], stage_ref, local_sem)
        cp.start()
        cp.wait()
        rdma = pltpu.make_async_remote_copy(
            src_ref=stage_ref,
            dst_ref=buf_ref.at[0],
            send_sem=send_sems.at[0],
            recv_sem=recv_sems.at[0],
            device_id=(right,),
            device_id_type=pl.DeviceIdType.MESH,
        )
        rdma.start()
        rdma.wait()

        for s in (1, 2):
            c = lax.rem(d - 1 - s + N_DEV, N_DEV)
            cp = pltpu.make_async_copy(
                part_ref.at[pl.ds(c * M_PER, M_PER), col], stage_ref, local_sem)
            cp.start()
            cp.wait()
            sbuf_ref[...] = buf_ref[s - 1] + stage_ref[...]
            rdma = pltpu.make_async_remote_copy(
                src_ref=sbuf_ref,
                dst_ref=buf_ref.at[s],
                send_sem=send_sems.at[s],
                recv_sem=recv_sems.at[s],
                device_id=(right,),
                device_id_type=pl.DeviceIdType.MESH,
            )
            rdma.start()
            rdma.wait()

        cp = pltpu.make_async_copy(
            part_ref.at[pl.ds(d * M_PER, M_PER), col], stage_ref, local_sem)
        cp.start()
        cp.wait()
        res_ref[...] = buf_ref[2] + stage_ref[...]
        amax_val = jnp.maximum(amax_val, jnp.max(jnp.abs(res_ref[...])))
        cp = pltpu.make_async_copy(res_ref, out_ref.at[:, col], local_sem)
        cp.start()
        cp.wait()

    axbuf_ref[3, :, :] = jnp.full((8, 128), amax_val, jnp.float32)
    sends = []
    for off in (1, 2, 3):
        tgt = lax.rem(d + off, N_DEV)
        rdma = pltpu.make_async_remote_copy(
            src_ref=axbuf_ref.at[3],
            dst_ref=axbuf_ref.at[off - 1],
            send_sem=ax_send_sems.at[off - 1],
            recv_sem=ax_recv_sems.at[off - 1],
            device_id=(tgt,),
            device_id_type=pl.DeviceIdType.MESH,
        )
        rdma.start()
        sends.append(rdma)
    for rdma in sends:
        rdma.wait_send()
    for rdma in sends:
        rdma.wait_recv()
    amax_ref[...] = jnp.max(axbuf_ref[...]).reshape(1, 1)


def _reduce_scatter_amax(partial):
    return pl.pallas_call(
        _rs_body,
        out_shape=[
            jax.ShapeDtypeStruct((M_PER, N_COLS), jnp.float32),
            jax.ShapeDtypeStruct((1, 1), jnp.float32),
        ],
        in_specs=[pl.BlockSpec(memory_space=pl.ANY)],
        out_specs=[
            pl.BlockSpec(memory_space=pl.ANY),
            pl.BlockSpec(memory_space=pltpu.VMEM),
        ],
        scratch_shapes=[
            pltpu.VMEM((3, M_PER, W), jnp.float32),
            pltpu.VMEM((M_PER, W), jnp.float32),
            pltpu.VMEM((M_PER, W), jnp.float32),
            pltpu.VMEM((M_PER, W), jnp.float32),
            pltpu.VMEM((N_DEV, 8, 128), jnp.float32),
            pltpu.SemaphoreType.DMA,
            pltpu.SemaphoreType.DMA((3,)),
            pltpu.SemaphoreType.DMA((3,)),
            pltpu.SemaphoreType.DMA((3,)),
            pltpu.SemaphoreType.DMA((3,)),
        ],
        compiler_params=pltpu.CompilerParams(collective_id=0),
    )(partial)


def kernel(x, w_mat):
    partial = jnp.dot(x, w_mat, precision=lax.Precision.HIGHEST)
    acc, amax = _reduce_scatter_amax(partial)
    scale = amax[0, 0] / 448.0
    q = (acc / scale).astype(jnp.float8_e4m3fn)
    q = lax.optimization_barrier(q)
    return q.astype(jnp.float32) * scale


# device time: 757812 ns/iter; 2.2378x vs baseline; 2.2378x over previous
import jax
import jax.numpy as jnp
from jax import lax
from jax.experimental import pallas as pl
from jax.experimental.pallas import tpu as pltpu

N_DEV = 4
M_PER = 1024
N_COLS = 8192
W = 1024
N_PAIRS = N_COLS // W // 2


def _rs_body(part_ref, out_ref, amax_ref,
             buf_cw, buf_ccw, stage_cw, stage_ccw, sbuf_cw, sbuf_ccw,
             axbuf_ref, local_sems,
             send_cw, recv_cw, send_ccw, recv_ccw, ax_send, ax_recv):
    d = lax.axis_index("i")
    left = lax.rem(d - 1 + N_DEV, N_DEV)
    right = lax.rem(d + 1, N_DEV)

    barrier = pltpu.get_barrier_semaphore()
    for nbr in (left, right):
        pl.semaphore_signal(
            barrier, inc=1,
            device_id=(nbr,), device_id_type=pl.DeviceIdType.MESH,
        )
    pl.semaphore_wait(barrier, 2)

    def stage_both(c_cw, col_cw, c_ccw, col_ccw):
        cp0 = pltpu.make_async_copy(
            part_ref.at[pl.ds(c_cw * M_PER, M_PER), col_cw],
            stage_cw, local_sems.at[0])
        cp1 = pltpu.make_async_copy(
            part_ref.at[pl.ds(c_ccw * M_PER, M_PER), col_ccw],
            stage_ccw, local_sems.at[1])
        cp0.start()
        cp1.start()
        cp0.wait()
        cp1.wait()

    amax_val = jnp.zeros((), jnp.float32)
    for p in range(N_PAIRS):
        col_cw = pl.ds(p * W, W)
        col_ccw = pl.ds((p + N_PAIRS) * W, W)

        for s in range(3):
            c_cw = lax.rem(d - 1 - s + 2 * N_DEV, N_DEV)
            c_ccw = lax.rem(d + 1 + s, N_DEV)
            stage_both(c_cw, col_cw, c_ccw, col_ccw)
            if s == 0:
                src_cw, src_ccw = stage_cw, stage_ccw
            else:
                sbuf_cw[...] = buf_cw[s - 1] + stage_cw[...]
                sbuf_ccw[...] = buf_ccw[s - 1] + stage_ccw[...]
                src_cw, src_ccw = sbuf_cw, sbuf_ccw
            rdma_cw = pltpu.make_async_remote_copy(
                src_ref=src_cw, dst_ref=buf_cw.at[s],
                send_sem=send_cw.at[s], recv_sem=recv_cw.at[s],
                device_id=(right,), device_id_type=pl.DeviceIdType.MESH,
            )
            rdma_ccw = pltpu.make_async_remote_copy(
                src_ref=src_ccw, dst_ref=buf_ccw.at[s],
                send_sem=send_ccw.at[s], recv_sem=recv_ccw.at[s],
                device_id=(left,), device_id_type=pl.DeviceIdType.MESH,
            )
            rdma_cw.start()
            rdma_ccw.start()
            rdma_cw.wait()
            rdma_ccw.wait()

        stage_both(d, col_cw, d, col_ccw)
        sbuf_cw[...] = buf_cw[2] + stage_cw[...]
        sbuf_ccw[...] = buf_ccw[2] + stage_ccw[...]
        amax_val = jnp.maximum(amax_val, jnp.max(jnp.abs(sbuf_cw[...])))
        amax_val = jnp.maximum(amax_val, jnp.max(jnp.abs(sbuf_ccw[...])))
        cp0 = pltpu.make_async_copy(sbuf_cw, out_ref.at[:, col_cw],
                                    local_sems.at[0])
        cp1 = pltpu.make_async_copy(sbuf_ccw, out_ref.at[:, col_ccw],
                                    local_sems.at[1])
        cp0.start()
        cp1.start()
        cp0.wait()
        cp1.wait()

    axbuf_ref[3, :, :] = jnp.full((8, 128), amax_val, jnp.float32)
    sends = []
    for off in (1, 2, 3):
        tgt = lax.rem(d + off, N_DEV)
        rdma = pltpu.make_async_remote_copy(
            src_ref=axbuf_ref.at[3],
            dst_ref=axbuf_ref.at[off - 1],
            send_sem=ax_send.at[off - 1],
            recv_sem=ax_recv.at[off - 1],
            device_id=(tgt,),
            device_id_type=pl.DeviceIdType.MESH,
        )
        rdma.start()
        sends.append(rdma)
    for rdma in sends:
        rdma.wait_send()
    for rdma in sends:
        rdma.wait_recv()
    amax_ref[...] = jnp.max(axbuf_ref[...]).reshape(1, 1)


def _reduce_scatter_amax(partial):
    return pl.pallas_call(
        _rs_body,
        out_shape=[
            jax.ShapeDtypeStruct((M_PER, N_COLS), jnp.float32),
            jax.ShapeDtypeStruct((1, 1), jnp.float32),
        ],
        in_specs=[pl.BlockSpec(memory_space=pl.ANY)],
        out_specs=[
            pl.BlockSpec(memory_space=pl.ANY),
            pl.BlockSpec(memory_space=pltpu.VMEM),
        ],
        scratch_shapes=[
            pltpu.VMEM((3, M_PER, W), jnp.float32),
            pltpu.VMEM((3, M_PER, W), jnp.float32),
            pltpu.VMEM((M_PER, W), jnp.float32),
            pltpu.VMEM((M_PER, W), jnp.float32),
            pltpu.VMEM((M_PER, W), jnp.float32),
            pltpu.VMEM((M_PER, W), jnp.float32),
            pltpu.VMEM((N_DEV, 8, 128), jnp.float32),
            pltpu.SemaphoreType.DMA((2,)),
            pltpu.SemaphoreType.DMA((3,)),
            pltpu.SemaphoreType.DMA((3,)),
            pltpu.SemaphoreType.DMA((3,)),
            pltpu.SemaphoreType.DMA((3,)),
            pltpu.SemaphoreType.DMA((3,)),
            pltpu.SemaphoreType.DMA((3,)),
        ],
        compiler_params=pltpu.CompilerParams(
            collective_id=0, vmem_limit_bytes=100 * 1024 * 1024),
    )(partial)


def kernel(x, w_mat):
    partial = jnp.dot(x, w_mat,
                      precision=lax.DotAlgorithmPreset.BF16_BF16_F32_X3)
    acc, amax = _reduce_scatter_amax(partial)
    scale = amax[0, 0] / 448.0
    q = (acc / scale).astype(jnp.float8_e4m3fn)
    q = lax.optimization_barrier(q)
    return q.astype(jnp.float32) * scale


# device time: 749970 ns/iter; 2.2612x vs baseline; 1.0105x over previous
import jax
import jax.numpy as jnp
from jax import lax
from jax.experimental import pallas as pl
from jax.experimental.pallas import tpu as pltpu

N_DEV = 4
M_PER = 1024
K_PER = 1024
N_COLS = 8192
W = 512
N_PAIRS = N_COLS // W // 2


def _mm(x_ref, c, wt):
    blk = x_ref[pl.ds(c * M_PER, M_PER), :]
    return jax.lax.dot_general(
        blk, wt, (((1,), (0,)), ((), ())),
        preferred_element_type=jnp.float32,
        precision=lax.Precision.HIGHEST,
    )


def _body(x_ref, w_ref, out_ref, amax_ref,
          buf_cw, buf_ccw, wt_cw, wt_ccw, sbuf_cw, sbuf_ccw,
          mmb_cw, mmb_ccw, res_cw, res_ccw, axbuf_ref,
          out_sems, wt_sems,
          send_cw, recv_cw, send_ccw, recv_ccw, ax_send, ax_recv):
    d = lax.axis_index("i")
    left = lax.rem(d - 1 + N_DEV, N_DEV)
    right = lax.rem(d + 1, N_DEV)

    barrier = pltpu.get_barrier_semaphore()
    for nbr in (left, right):
        pl.semaphore_signal(
            barrier, inc=1,
            device_id=(nbr,), device_id_type=pl.DeviceIdType.MESH,
        )
    pl.semaphore_wait(barrier, 2)

    def wt_fetch(p, slot):
        cw = pltpu.make_async_copy(
            w_ref.at[:, pl.ds(p * W, W)], wt_cw.at[slot], wt_sems.at[0])
        ccw = pltpu.make_async_copy(
            w_ref.at[:, pl.ds((p + N_PAIRS) * W, W)], wt_ccw.at[slot],
            wt_sems.at[1])
        cw.start()
        ccw.start()
        return cw, ccw

    wt_fetch(0, 0)

    def wt_wait(slot):
        pltpu.make_async_copy(
            w_ref.at[:, pl.ds(0, W)], wt_cw.at[slot], wt_sems.at[0]).wait()
        pltpu.make_async_copy(
            w_ref.at[:, pl.ds(0, W)], wt_ccw.at[slot], wt_sems.at[1]).wait()

    def out_wait():
        pltpu.make_async_copy(res_cw, out_ref.at[:, pl.ds(0, W)],
                              out_sems.at[0]).wait()
        pltpu.make_async_copy(res_ccw, out_ref.at[:, pl.ds(0, W)],
                              out_sems.at[1]).wait()

    def pair_body(p, amax_val):
        slot = lax.rem(p, 2)
        col_cw = pl.ds(p * W, W)
        col_ccw = pl.ds((p + N_PAIRS) * W, W)
        wcw = wt_cw.at[slot]
        wccw = wt_ccw.at[slot]
        wt_wait(slot)

        for s in range(3):
            if s == 0:
                sbuf_cw[...] = _mm(x_ref, lax.rem(d - 1 + N_DEV, N_DEV), wcw[...])
                sbuf_ccw[...] = _mm(x_ref, lax.rem(d + 1, N_DEV), wccw[...])
            else:
                sbuf_cw[...] = buf_cw[s - 1] + mmb_cw[...]
                sbuf_ccw[...] = buf_ccw[s - 1] + mmb_ccw[...]
            rdma_cw = pltpu.make_async_remote_copy(
                src_ref=sbuf_cw, dst_ref=buf_cw.at[s],
                send_sem=send_cw.at[s], recv_sem=recv_cw.at[s],
                device_id=(right,), device_id_type=pl.DeviceIdType.MESH,
            )
            rdma_ccw = pltpu.make_async_remote_copy(
                src_ref=sbuf_ccw, dst_ref=buf_ccw.at[s],
                send_sem=send_ccw.at[s], recv_sem=recv_ccw.at[s],
                device_id=(left,), device_id_type=pl.DeviceIdType.MESH,
            )
            rdma_cw.start()
            rdma_ccw.start()
            if s == 0:
                @pl.when(p + 1 < N_PAIRS)
                def _():
                    wt_fetch(p + 1, lax.rem(p + 1, 2))
            c_cw = lax.rem(d - 2 - s + 2 * N_DEV, N_DEV)
            c_ccw = lax.rem(d + 2 + s, N_DEV)
            mmb_cw[...] = _mm(x_ref, c_cw, wcw[...])
            mmb_ccw[...] = _mm(x_ref, c_ccw, wccw[...])
            rdma_cw.wait()
            rdma_ccw.wait()

        @pl.when(p > 0)
        def _():
            out_wait()
        res_cw[...] = buf_cw[2] + mmb_cw[...]
        res_ccw[...] = buf_ccw[2] + mmb_ccw[...]
        amax_val = jnp.maximum(amax_val, jnp.max(jnp.abs(res_cw[...])))
        amax_val = jnp.maximum(amax_val, jnp.max(jnp.abs(res_ccw[...])))
        pltpu.make_async_copy(res_cw, out_ref.at[:, col_cw],
                              out_sems.at[0]).start()
        pltpu.make_async_copy(res_ccw, out_ref.at[:, col_ccw],
                              out_sems.at[1]).start()
        return amax_val

    amax_val = lax.fori_loop(0, N_PAIRS, pair_body,
                             jnp.zeros((), jnp.float32))
    out_wait()

    axbuf_ref[3, :, :] = jnp.full((8, 128), amax_val, jnp.float32)
    sends = []
    for off in (1, 2, 3):
        tgt = lax.rem(d + off, N_DEV)
        rdma = pltpu.make_async_remote_copy(
            src_ref=axbuf_ref.at[3],
            dst_ref=axbuf_ref.at[off - 1],
            send_sem=ax_send.at[off - 1],
            recv_sem=ax_recv.at[off - 1],
            device_id=(tgt,),
            device_id_type=pl.DeviceIdType.MESH,
        )
        rdma.start()
        sends.append(rdma)
    for rdma in sends:
        rdma.wait_send()
    for rdma in sends:
        rdma.wait_recv()
    amax_ref[...] = jnp.max(axbuf_ref[...]).reshape(1, 1)


def _gemm_rs_amax(x, w_mat):
    return pl.pallas_call(
        _body,
        out_shape=[
            jax.ShapeDtypeStruct((M_PER, N_COLS), jnp.float32),
            jax.ShapeDtypeStruct((1, 1), jnp.float32),
        ],
        in_specs=[
            pl.BlockSpec(memory_space=pltpu.VMEM),
            pl.BlockSpec(memory_space=pl.ANY),
        ],
        out_specs=[
            pl.BlockSpec(memory_space=pl.ANY),
            pl.BlockSpec(memory_space=pltpu.VMEM),
        ],
        scratch_shapes=[
            pltpu.VMEM((3, M_PER, W), jnp.float32),
            pltpu.VMEM((3, M_PER, W), jnp.float32),
            pltpu.VMEM((2, K_PER, W), jnp.float32),
            pltpu.VMEM((2, K_PER, W), jnp.float32),
            pltpu.VMEM((M_PER, W), jnp.float32),
            pltpu.VMEM((M_PER, W), jnp.float32),
            pltpu.VMEM((M_PER, W), jnp.float32),
            pltpu.VMEM((M_PER, W), jnp.float32),
            pltpu.VMEM((M_PER, W), jnp.float32),
            pltpu.VMEM((M_PER, W), jnp.float32),
            pltpu.VMEM((N_DEV, 8, 128), jnp.float32),
            pltpu.SemaphoreType.DMA((2,)),
            pltpu.SemaphoreType.DMA((2,)),
            pltpu.SemaphoreType.DMA((3,)),
            pltpu.SemaphoreType.DMA((3,)),
            pltpu.SemaphoreType.DMA((3,)),
            pltpu.SemaphoreType.DMA((3,)),
            pltpu.SemaphoreType.DMA((3,)),
            pltpu.SemaphoreType.DMA((3,)),
        ],
        compiler_params=pltpu.CompilerParams(
            collective_id=0, vmem_limit_bytes=63 * 1024 * 1024),
    )(x, w_mat)


def kernel(x, w_mat):
    acc, amax = _gemm_rs_amax(x, w_mat)
    scale = amax[0, 0] / 448.0
    q = (acc / scale).astype(jnp.float8_e4m3fn)
    q = lax.optimization_barrier(q)
    return q.astype(jnp.float32) * scale


# device time: 674238 ns/iter; 2.5151x vs baseline; 1.1123x over previous
import jax
import jax.numpy as jnp
from jax import lax
from jax.experimental import pallas as pl
from jax.experimental.pallas import tpu as pltpu

N_DEV = 4
M_PER = 1024
K_PER = 1024
N_COLS = 8192
W = 512
N_PAIRS = N_COLS // W // 2


def _mm(x_ref, c, wt):
    blk = x_ref[pl.ds(c * M_PER, M_PER), :]
    return jax.lax.dot_general(
        blk, wt, (((1,), (0,)), ((), ())),
        preferred_element_type=jnp.float32,
        precision=lax.Precision.HIGHEST,
    )


def _body(x_ref, w_ref, out_ref, amax_ref,
          buf_cw, buf_ccw, wt_cw, wt_ccw, sbuf_cw, sbuf_ccw,
          mmb_cw, mmb_ccw, pre_cw, pre_ccw, res_cw, res_ccw, axbuf_ref,
          out_sems, wt_sems,
          send_cw, recv_cw, send_ccw, recv_ccw, ax_send, ax_recv):
    d = lax.axis_index("i")
    left = lax.rem(d - 1 + N_DEV, N_DEV)
    right = lax.rem(d + 1, N_DEV)

    barrier = pltpu.get_barrier_semaphore()
    for nbr in (left, right):
        pl.semaphore_signal(
            barrier, inc=1,
            device_id=(nbr,), device_id_type=pl.DeviceIdType.MESH,
        )
    pl.semaphore_wait(barrier, 2)

    def wt_fetch(p, slot):
        cw = pltpu.make_async_copy(
            w_ref.at[:, pl.ds(p * W, W)], wt_cw.at[slot], wt_sems.at[0])
        ccw = pltpu.make_async_copy(
            w_ref.at[:, pl.ds((p + N_PAIRS) * W, W)], wt_ccw.at[slot],
            wt_sems.at[1])
        cw.start()
        ccw.start()
        return cw, ccw

    def wt_wait(slot):
        pltpu.make_async_copy(
            w_ref.at[:, pl.ds(0, W)], wt_cw.at[slot], wt_sems.at[0]).wait()
        pltpu.make_async_copy(
            w_ref.at[:, pl.ds(0, W)], wt_ccw.at[slot], wt_sems.at[1]).wait()

    def out_wait():
        pltpu.make_async_copy(res_cw, out_ref.at[:, pl.ds(0, W)],
                              out_sems.at[0]).wait()
        pltpu.make_async_copy(res_ccw, out_ref.at[:, pl.ds(0, W)],
                              out_sems.at[1]).wait()

    def pair_body(p, amax_val):
        slot = lax.rem(p, 2)
        col_cw = pl.ds(p * W, W)
        col_ccw = pl.ds((p + N_PAIRS) * W, W)
        wcw = wt_cw.at[slot]
        wccw = wt_ccw.at[slot]

        for s in range(3):
            if s == 0:
                src_cw, src_ccw = pre_cw, pre_ccw
            else:
                sbuf_cw[...] = buf_cw[s - 1] + mmb_cw[...]
                sbuf_ccw[...] = buf_ccw[s - 1] + mmb_ccw[...]
                src_cw, src_ccw = sbuf_cw, sbuf_ccw
            rdma_cw = pltpu.make_async_remote_copy(
                src_ref=src_cw, dst_ref=buf_cw.at[s],
                send_sem=send_cw.at[s], recv_sem=recv_cw.at[s],
                device_id=(right,), device_id_type=pl.DeviceIdType.MESH,
            )
            rdma_ccw = pltpu.make_async_remote_copy(
                src_ref=src_ccw, dst_ref=buf_ccw.at[s],
                send_sem=send_ccw.at[s], recv_sem=recv_ccw.at[s],
                device_id=(left,), device_id_type=pl.DeviceIdType.MESH,
            )
            rdma_cw.start()
            rdma_ccw.start()
            if s == 0:
                @pl.when(p + 1 < N_PAIRS)
                def _():
                    wt_fetch(p + 1, lax.rem(p + 1, 2))
            c_cw = lax.rem(d - 2 - s + 2 * N_DEV, N_DEV)
            c_ccw = lax.rem(d + 2 + s, N_DEV)
            mmb_cw[...] = _mm(x_ref, c_cw, wcw[...])
            mmb_ccw[...] = _mm(x_ref, c_ccw, wccw[...])
            if s == 2:
                @pl.when(p + 1 < N_PAIRS)
                def _():
                    nslot = lax.rem(p + 1, 2)
                    wt_wait(nslot)
                    pre_cw[...] = _mm(x_ref, lax.rem(d - 1 + N_DEV, N_DEV),
                                      wt_cw[nslot])
                    pre_ccw[...] = _mm(x_ref, lax.rem(d + 1, N_DEV),
                                       wt_ccw[nslot])
            rdma_cw.wait()
            rdma_ccw.wait()

        @pl.when(p > 0)
        def _():
            out_wait()
        res_cw[...] = buf_cw[2] + mmb_cw[...]
        res_ccw[...] = buf_ccw[2] + mmb_ccw[...]
        amax_val = jnp.maximum(amax_val, jnp.max(jnp.abs(res_cw[...])))
        amax_val = jnp.maximum(amax_val, jnp.max(jnp.abs(res_ccw[...])))
        pltpu.make_async_copy(res_cw, out_ref.at[:, col_cw],
                              out_sems.at[0]).start()
        pltpu.make_async_copy(res_ccw, out_ref.at[:, col_ccw],
                              out_sems.at[1]).start()
        return amax_val

    wt_fetch(0, 0)
    wt_wait(0)
    pre_cw[...] = _mm(x_ref, lax.rem(d - 1 + N_DEV, N_DEV), wt_cw[0])
    pre_ccw[...] = _mm(x_ref, lax.rem(d + 1, N_DEV), wt_ccw[0])

    amax_val = lax.fori_loop(0, N_PAIRS, pair_body,
                             jnp.zeros((), jnp.float32))
    out_wait()

    axbuf_ref[3, :, :] = jnp.full((8, 128), amax_val, jnp.float32)
    sends = []
    for off in (1, 2, 3):
        tgt = lax.rem(d + off, N_DEV)
        rdma = pltpu.make_async_remote_copy(
            src_ref=axbuf_ref.at[3],
            dst_ref=axbuf_ref.at[off - 1],
            send_sem=ax_send.at[off - 1],
            recv_sem=ax_recv.at[off - 1],
            device_id=(tgt,),
            device_id_type=pl.DeviceIdType.MESH,
        )
        rdma.start()
        sends.append(rdma)
    for rdma in sends:
        rdma.wait_send()
    for rdma in sends:
        rdma.wait_recv()
    amax_ref[...] = jnp.max(axbuf_ref[...]).reshape(1, 1)


def _gemm_rs_amax(x, w_mat):
    return pl.pallas_call(
        _body,
        out_shape=[
            jax.ShapeDtypeStruct((M_PER, N_COLS), jnp.float32),
            jax.ShapeDtypeStruct((1, 1), jnp.float32),
        ],
        in_specs=[
            pl.BlockSpec(memory_space=pltpu.VMEM),
            pl.BlockSpec(memory_space=pl.ANY),
        ],
        out_specs=[
            pl.BlockSpec(memory_space=pl.ANY),
            pl.BlockSpec(memory_space=pltpu.VMEM),
        ],
        scratch_shapes=[
            pltpu.VMEM((3, M_PER, W), jnp.float32),
            pltpu.VMEM((3, M_PER, W), jnp.float32),
            pltpu.VMEM((2, K_PER, W), jnp.float32),
            pltpu.VMEM((2, K_PER, W), jnp.float32),
            pltpu.VMEM((M_PER, W), jnp.float32),
            pltpu.VMEM((M_PER, W), jnp.float32),
            pltpu.VMEM((M_PER, W), jnp.float32),
            pltpu.VMEM((M_PER, W), jnp.float32),
            pltpu.VMEM((M_PER, W), jnp.float32),
            pltpu.VMEM((M_PER, W), jnp.float32),
            pltpu.VMEM((M_PER, W), jnp.float32),
            pltpu.VMEM((M_PER, W), jnp.float32),
            pltpu.VMEM((N_DEV, 8, 128), jnp.float32),
            pltpu.SemaphoreType.DMA((2,)),
            pltpu.SemaphoreType.DMA((2,)),
            pltpu.SemaphoreType.DMA((3,)),
            pltpu.SemaphoreType.DMA((3,)),
            pltpu.SemaphoreType.DMA((3,)),
            pltpu.SemaphoreType.DMA((3,)),
            pltpu.SemaphoreType.DMA((3,)),
            pltpu.SemaphoreType.DMA((3,)),
        ],
        compiler_params=pltpu.CompilerParams(
            collective_id=0,
            vmem_limit_bytes=int(63.9 * 1024 * 1024)),
    )(x, w_mat)


def kernel(x, w_mat):
    acc, amax = _gemm_rs_amax(x, w_mat)
    scale = amax[0, 0] / 448.0
    q = (acc / scale).astype(jnp.float8_e4m3fn)
    q = lax.optimization_barrier(q)
    return q.astype(jnp.float32) * scale


# device time: 654491 ns/iter; 2.5910x vs baseline; 1.0302x over previous
import jax
import jax.numpy as jnp
from jax import lax
from jax.experimental import pallas as pl
from jax.experimental.pallas import tpu as pltpu

N_DEV = 4
M_PER = 1024
K_PER = 1024
N_COLS = 8192
W = 512
N_PAIRS = N_COLS // W // 2


def _mm(x_ref, c, wt):
    blk = x_ref[pl.ds(c * M_PER, M_PER), :]
    return jax.lax.dot_general(
        blk, wt, (((1,), (0,)), ((), ())),
        preferred_element_type=jnp.float32,
        precision=lax.Precision.HIGHEST,
    )


def _body(x_ref, w_ref, out_ref, amax_ref,
          buf_cw, buf_ccw, wt_cw, wt_ccw, sbuf_cw, sbuf_ccw,
          mmb_cw, mmb_ccw, pre_cw, pre_ccw, res_cw, res_ccw, axbuf_ref,
          out_sems, wt_sems,
          send_cw, recv_cw, send_ccw, recv_ccw, ax_send, ax_recv):
    d = lax.axis_index("i")
    left = lax.rem(d - 1 + N_DEV, N_DEV)
    right = lax.rem(d + 1, N_DEV)

    barrier = pltpu.get_barrier_semaphore()
    for nbr in (left, right):
        pl.semaphore_signal(
            barrier, inc=1,
            device_id=(nbr,), device_id_type=pl.DeviceIdType.MESH,
        )
    pl.semaphore_wait(barrier, 2)

    def wt_fetch(p, slot):
        cw = pltpu.make_async_copy(
            w_ref.at[:, pl.ds(p * W, W)], wt_cw.at[slot], wt_sems.at[0])
        ccw = pltpu.make_async_copy(
            w_ref.at[:, pl.ds((p + N_PAIRS) * W, W)], wt_ccw.at[slot],
            wt_sems.at[1])
        cw.start()
        ccw.start()
        return cw, ccw

    def wt_wait(slot):
        pltpu.make_async_copy(
            w_ref.at[:, pl.ds(0, W)], wt_cw.at[slot], wt_sems.at[0]).wait()
        pltpu.make_async_copy(
            w_ref.at[:, pl.ds(0, W)], wt_ccw.at[slot], wt_sems.at[1]).wait()

    def out_wait():
        pltpu.make_async_copy(res_cw, out_ref.at[:, pl.ds(0, W)],
                              out_sems.at[0]).wait()
        pltpu.make_async_copy(res_ccw, out_ref.at[:, pl.ds(0, W)],
                              out_sems.at[1]).wait()

    def pair_body(p, amax_val):
        slot = lax.rem(p, 2)
        col_cw = pl.ds(p * W, W)
        col_ccw = pl.ds((p + N_PAIRS) * W, W)
        wcw = wt_cw.at[slot]
        wccw = wt_ccw.at[slot]

        for s in range(3):
            if s == 0:
                src_cw, src_ccw = pre_cw, pre_ccw
            else:
                sbuf_cw[...] = buf_cw[s - 1] + mmb_cw[...]
                sbuf_ccw[...] = buf_ccw[s - 1] + mmb_ccw[...]
                src_cw, src_ccw = sbuf_cw, sbuf_ccw
            rdma_cw = pltpu.make_async_remote_copy(
                src_ref=src_cw, dst_ref=buf_cw.at[s],
                send_sem=send_cw.at[s], recv_sem=recv_cw.at[s],
                device_id=(right,), device_id_type=pl.DeviceIdType.MESH,
            )
            rdma_ccw = pltpu.make_async_remote_copy(
                src_ref=src_ccw, dst_ref=buf_ccw.at[s],
                send_sem=send_ccw.at[s], recv_sem=recv_ccw.at[s],
                device_id=(left,), device_id_type=pl.DeviceIdType.MESH,
            )
            rdma_cw.start()
            rdma_ccw.start()
            if s == 0:
                @pl.when(p + 1 < N_PAIRS)
                def _():
                    wt_fetch(p + 1, lax.rem(p + 1, 2))
            c_cw = lax.rem(d - 2 - s + 2 * N_DEV, N_DEV)
            c_ccw = lax.rem(d + 2 + s, N_DEV)
            mmb_cw[...] = _mm(x_ref, c_cw, wcw[...])
            mmb_ccw[...] = _mm(x_ref, c_ccw, wccw[...])
            if s == 1:
                @pl.when(p + 1 < N_PAIRS)
                def _():
                    nslot = lax.rem(p + 1, 2)
                    wt_wait(nslot)
                    pre_cw[...] = _mm(x_ref, lax.rem(d - 1 + N_DEV, N_DEV),
                                      wt_cw[nslot])
            if s == 2:
                @pl.when(p + 1 < N_PAIRS)
                def _():
                    nslot = lax.rem(p + 1, 2)
                    pre_ccw[...] = _mm(x_ref, lax.rem(d + 1, N_DEV),
                                       wt_ccw[nslot])
            rdma_cw.wait()
            rdma_ccw.wait()

        @pl.when(p > 0)
        def _():
            out_wait()
        res_cw[...] = buf_cw[2] + mmb_cw[...]
        res_ccw[...] = buf_ccw[2] + mmb_ccw[...]
        amax_val = jnp.maximum(amax_val, jnp.max(jnp.abs(res_cw[...])))
        amax_val = jnp.maximum(amax_val, jnp.max(jnp.abs(res_ccw[...])))
        pltpu.make_async_copy(res_cw, out_ref.at[:, col_cw],
                              out_sems.at[0]).start()
        pltpu.make_async_copy(res_ccw, out_ref.at[:, col_ccw],
                              out_sems.at[1]).start()
        return amax_val

    wt_fetch(0, 0)
    wt_wait(0)
    pre_cw[...] = _mm(x_ref, lax.rem(d - 1 + N_DEV, N_DEV), wt_cw[0])
    pre_ccw[...] = _mm(x_ref, lax.rem(d + 1, N_DEV), wt_ccw[0])

    amax_val = lax.fori_loop(0, N_PAIRS, pair_body,
                             jnp.zeros((), jnp.float32))
    out_wait()

    axbuf_ref[3, :, :] = jnp.full((8, 128), amax_val, jnp.float32)
    sends = []
    for off in (1, 2, 3):
        tgt = lax.rem(d + off, N_DEV)
        rdma = pltpu.make_async_remote_copy(
            src_ref=axbuf_ref.at[3],
            dst_ref=axbuf_ref.at[off - 1],
            send_sem=ax_send.at[off - 1],
            recv_sem=ax_recv.at[off - 1],
            device_id=(tgt,),
            device_id_type=pl.DeviceIdType.MESH,
        )
        rdma.start()
        sends.append(rdma)
    for rdma in sends:
        rdma.wait_send()
    for rdma in sends:
        rdma.wait_recv()
    amax_ref[...] = jnp.max(axbuf_ref[...]).reshape(1, 1)


def _gemm_rs_amax(x, w_mat):
    return pl.pallas_call(
        _body,
        out_shape=[
            jax.ShapeDtypeStruct((M_PER, N_COLS), jnp.float32),
            jax.ShapeDtypeStruct((1, 1), jnp.float32),
        ],
        in_specs=[
            pl.BlockSpec(memory_space=pltpu.VMEM),
            pl.BlockSpec(memory_space=pl.ANY),
        ],
        out_specs=[
            pl.BlockSpec(memory_space=pl.ANY),
            pl.BlockSpec(memory_space=pltpu.VMEM),
        ],
        scratch_shapes=[
            pltpu.VMEM((3, M_PER, W), jnp.float32),
            pltpu.VMEM((3, M_PER, W), jnp.float32),
            pltpu.VMEM((2, K_PER, W), jnp.float32),
            pltpu.VMEM((2, K_PER, W), jnp.float32),
            pltpu.VMEM((M_PER, W), jnp.float32),
            pltpu.VMEM((M_PER, W), jnp.float32),
            pltpu.VMEM((M_PER, W), jnp.float32),
            pltpu.VMEM((M_PER, W), jnp.float32),
            pltpu.VMEM((M_PER, W), jnp.float32),
            pltpu.VMEM((M_PER, W), jnp.float32),
            pltpu.VMEM((M_PER, W), jnp.float32),
            pltpu.VMEM((M_PER, W), jnp.float32),
            pltpu.VMEM((N_DEV, 8, 128), jnp.float32),
            pltpu.SemaphoreType.DMA((2,)),
            pltpu.SemaphoreType.DMA((2,)),
            pltpu.SemaphoreType.DMA((3,)),
            pltpu.SemaphoreType.DMA((3,)),
            pltpu.SemaphoreType.DMA((3,)),
            pltpu.SemaphoreType.DMA((3,)),
            pltpu.SemaphoreType.DMA((3,)),
            pltpu.SemaphoreType.DMA((3,)),
        ],
        compiler_params=pltpu.CompilerParams(
            collective_id=0,
            vmem_limit_bytes=int(63.9 * 1024 * 1024)),
    )(x, w_mat)


def kernel(x, w_mat):
    acc, amax = _gemm_rs_amax(x, w_mat)
    scale = amax[0, 0] / 448.0
    q = (acc / scale).astype(jnp.float8_e4m3fn)
    q = lax.optimization_barrier(q)
    return q.astype(jnp.float32) * scale


# device time: 654099 ns/iter; 2.5926x vs baseline; 1.0006x over previous
import jax
import jax.numpy as jnp
from jax import lax
from jax.experimental import pallas as pl
from jax.experimental.pallas import tpu as pltpu

N_DEV = 4
M_PER = 1024
K_PER = 1024
N_COLS = 8192
W = 512
N_PAIRS = N_COLS // W // 2


def _mm(x_ref, c, wt):
    blk = x_ref[pl.ds(c * M_PER, M_PER), :]
    return jax.lax.dot_general(
        blk, wt, (((1,), (0,)), ((), ())),
        preferred_element_type=jnp.float32,
        precision=lax.Precision.HIGHEST,
    )


def _body(x_ref, w_ref, out_ref, amax_ref,
          buf_cw, buf_ccw, wt_cw, wt_ccw, sbuf_cw, sbuf_ccw,
          mmb_cw, mmb_ccw, pre_cw, pre_ccw, res_cw, res_ccw, axbuf_ref,
          out_sems, wt_sems,
          send_cw, recv_cw, send_ccw, recv_ccw, ax_send, ax_recv):
    d = lax.axis_index("i")
    left = lax.rem(d - 1 + N_DEV, N_DEV)
    right = lax.rem(d + 1, N_DEV)

    def wt_fetch(p, slot):
        cw = pltpu.make_async_copy(
            w_ref.at[:, pl.ds(p * W, W)], wt_cw.at[slot], wt_sems.at[0])
        ccw = pltpu.make_async_copy(
            w_ref.at[:, pl.ds((p + N_PAIRS) * W, W)], wt_ccw.at[slot],
            wt_sems.at[1])
        cw.start()
        ccw.start()
        return cw, ccw

    def wt_wait(slot):
        pltpu.make_async_copy(
            w_ref.at[:, pl.ds(0, W)], wt_cw.at[slot], wt_sems.at[0]).wait()
        pltpu.make_async_copy(
            w_ref.at[:, pl.ds(0, W)], wt_ccw.at[slot], wt_sems.at[1]).wait()

    def out_wait():
        pltpu.make_async_copy(res_cw, out_ref.at[:, pl.ds(0, W)],
                              out_sems.at[0]).wait()
        pltpu.make_async_copy(res_ccw, out_ref.at[:, pl.ds(0, W)],
                              out_sems.at[1]).wait()

    def pair_body(p, amax_val):
        slot = lax.rem(p, 2)
        col_cw = pl.ds(p * W, W)
        col_ccw = pl.ds((p + N_PAIRS) * W, W)
        wcw = wt_cw.at[slot]
        wccw = wt_ccw.at[slot]

        for s in range(3):
            if s == 0:
                src_cw, src_ccw = pre_cw, pre_ccw
            else:
                sbuf_cw[...] = buf_cw[s - 1] + mmb_cw[...]
                sbuf_ccw[...] = buf_ccw[s - 1] + mmb_ccw[...]
                src_cw, src_ccw = sbuf_cw, sbuf_ccw
            rdma_cw = pltpu.make_async_remote_copy(
                src_ref=src_cw, dst_ref=buf_cw.at[s],
                send_sem=send_cw.at[s], recv_sem=recv_cw.at[s],
                device_id=(right,), device_id_type=pl.DeviceIdType.MESH,
            )
            rdma_ccw = pltpu.make_async_remote_copy(
                src_ref=src_ccw, dst_ref=buf_ccw.at[s],
                send_sem=send_ccw.at[s], recv_sem=recv_ccw.at[s],
                device_id=(left,), device_id_type=pl.DeviceIdType.MESH,
            )
            rdma_cw.start()
            rdma_ccw.start()
            if s == 0:
                @pl.when(p + 1 < N_PAIRS)
                def _():
                    wt_fetch(p + 1, lax.rem(p + 1, 2))
            c_cw = lax.rem(d - 2 - s + 2 * N_DEV, N_DEV)
            c_ccw = lax.rem(d + 2 + s, N_DEV)
            mmb_cw[...] = _mm(x_ref, c_cw, wcw[...])
            mmb_ccw[...] = _mm(x_ref, c_ccw, wccw[...])
            if s == 1:
                @pl.when(p + 1 < N_PAIRS)
                def _():
                    nslot = lax.rem(p + 1, 2)
                    wt_wait(nslot)
                    pre_cw[...] = _mm(x_ref, lax.rem(d - 1 + N_DEV, N_DEV),
                                      wt_cw[nslot])
            if s == 2:
                @pl.when(p + 1 < N_PAIRS)
                def _():
                    nslot = lax.rem(p + 1, 2)
                    pre_ccw[...] = _mm(x_ref, lax.rem(d + 1, N_DEV),
                                       wt_ccw[nslot])
            rdma_cw.wait()
            rdma_ccw.wait()

        @pl.when(p > 0)
        def _():
            out_wait()
        res_cw[...] = buf_cw[2] + mmb_cw[...]
        res_ccw[...] = buf_ccw[2] + mmb_ccw[...]
        amax_val = jnp.maximum(amax_val, jnp.max(jnp.abs(res_cw[...])))
        amax_val = jnp.maximum(amax_val, jnp.max(jnp.abs(res_ccw[...])))
        pltpu.make_async_copy(res_cw, out_ref.at[:, col_cw],
                              out_sems.at[0]).start()
        pltpu.make_async_copy(res_ccw, out_ref.at[:, col_ccw],
                              out_sems.at[1]).start()
        return amax_val

    wt_fetch(0, 0)
    wt_wait(0)
    pre_cw[...] = _mm(x_ref, lax.rem(d - 1 + N_DEV, N_DEV), wt_cw[0])
    pre_ccw[...] = _mm(x_ref, lax.rem(d + 1, N_DEV), wt_ccw[0])

    barrier = pltpu.get_barrier_semaphore()
    for nbr in (left, right):
        pl.semaphore_signal(
            barrier, inc=1,
            device_id=(nbr,), device_id_type=pl.DeviceIdType.MESH,
        )
    pl.semaphore_wait(barrier, 2)

    amax_val = lax.fori_loop(0, N_PAIRS, pair_body,
                             jnp.zeros((), jnp.float32))

    axbuf_ref[3, :, :] = jnp.full((8, 128), amax_val, jnp.float32)
    sends = []
    for off in (1, 2, 3):
        tgt = lax.rem(d + off, N_DEV)
        rdma = pltpu.make_async_remote_copy(
            src_ref=axbuf_ref.at[3],
            dst_ref=axbuf_ref.at[off - 1],
            send_sem=ax_send.at[off - 1],
            recv_sem=ax_recv.at[off - 1],
            device_id=(tgt,),
            device_id_type=pl.DeviceIdType.MESH,
        )
        rdma.start()
        sends.append(rdma)
    out_wait()
    for rdma in sends:
        rdma.wait_send()
    for rdma in sends:
        rdma.wait_recv()
    amax_ref[...] = jnp.max(axbuf_ref[...]).reshape(1, 1)


def _gemm_rs_amax(x, w_mat):
    return pl.pallas_call(
        _body,
        out_shape=[
            jax.ShapeDtypeStruct((M_PER, N_COLS), jnp.float32),
            jax.ShapeDtypeStruct((1, 1), jnp.float32),
        ],
        in_specs=[
            pl.BlockSpec(memory_space=pltpu.VMEM),
            pl.BlockSpec(memory_space=pl.ANY),
        ],
        out_specs=[
            pl.BlockSpec(memory_space=pl.ANY),
            pl.BlockSpec(memory_space=pltpu.VMEM),
        ],
        scratch_shapes=[
            pltpu.VMEM((3, M_PER, W), jnp.float32),
            pltpu.VMEM((3, M_PER, W), jnp.float32),
            pltpu.VMEM((2, K_PER, W), jnp.float32),
            pltpu.VMEM((2, K_PER, W), jnp.float32),
            pltpu.VMEM((M_PER, W), jnp.float32),
            pltpu.VMEM((M_PER, W), jnp.float32),
            pltpu.VMEM((M_PER, W), jnp.float32),
            pltpu.VMEM((M_PER, W), jnp.float32),
            pltpu.VMEM((M_PER, W), jnp.float32),
            pltpu.VMEM((M_PER, W), jnp.float32),
            pltpu.VMEM((M_PER, W), jnp.float32),
            pltpu.VMEM((M_PER, W), jnp.float32),
            pltpu.VMEM((N_DEV, 8, 128), jnp.float32),
            pltpu.SemaphoreType.DMA((2,)),
            pltpu.SemaphoreType.DMA((2,)),
            pltpu.SemaphoreType.DMA((3,)),
            pltpu.SemaphoreType.DMA((3,)),
            pltpu.SemaphoreType.DMA((3,)),
            pltpu.SemaphoreType.DMA((3,)),
            pltpu.SemaphoreType.DMA((3,)),
            pltpu.SemaphoreType.DMA((3,)),
        ],
        compiler_params=pltpu.CompilerParams(
            collective_id=0,
            vmem_limit_bytes=int(63.9 * 1024 * 1024)),
    )(x, w_mat)


def kernel(x, w_mat):
    acc, amax = _gemm_rs_amax(x, w_mat)
    scale = amax[0, 0] / 448.0
    q = (acc / scale).astype(jnp.float8_e4m3fn)
    q = lax.optimization_barrier(q)
    return q.astype(jnp.float32) * scale


# device time: 636308 ns/iter; 2.6651x vs baseline; 1.0280x over previous
import jax
import jax.numpy as jnp
from jax import lax
from jax.experimental import pallas as pl
from jax.experimental.pallas import tpu as pltpu

N_DEV = 4
M_PER = 1024
K_PER = 1024
N_COLS = 8192
W = 512
N_PAIRS = N_COLS // W // 2


def _dot(a, b):
    return jax.lax.dot_general(
        a, b, (((1,), (0,)), ((), ())),
        preferred_element_type=jnp.float32,
    )


def _mm(x_ref, c, wt):
    blk = x_ref[pl.ds(c * M_PER, M_PER), :]
    a_hi = blk.astype(jnp.bfloat16)
    a_lo = (blk - a_hi.astype(jnp.float32)).astype(jnp.bfloat16)
    b_hi = wt.astype(jnp.bfloat16)
    b_lo = (wt - b_hi.astype(jnp.float32)).astype(jnp.bfloat16)
    return _dot(a_hi, b_hi) + _dot(a_hi, b_lo) + _dot(a_lo, b_hi)


def _body(x_ref, w_ref, out_ref, amax_ref,
          buf_cw, buf_ccw, wt_cw, wt_ccw, sbuf_cw, sbuf_ccw,
          mmb_cw, mmb_ccw, pre_cw, pre_ccw, res_cw, res_ccw, axbuf_ref,
          out_sems, wt_sems,
          send_cw, recv_cw, send_ccw, recv_ccw, ax_send, ax_recv):
    d = lax.axis_index("i")
    left = lax.rem(d - 1 + N_DEV, N_DEV)
    right = lax.rem(d + 1, N_DEV)

    def wt_fetch(p, slot):
        cw = pltpu.make_async_copy(
            w_ref.at[:, pl.ds(p * W, W)], wt_cw.at[slot], wt_sems.at[0])
        ccw = pltpu.make_async_copy(
            w_ref.at[:, pl.ds((p + N_PAIRS) * W, W)], wt_ccw.at[slot],
            wt_sems.at[1])
        cw.start()
        ccw.start()
        return cw, ccw

    def wt_wait(slot):
        pltpu.make_async_copy(
            w_ref.at[:, pl.ds(0, W)], wt_cw.at[slot], wt_sems.at[0]).wait()
        pltpu.make_async_copy(
            w_ref.at[:, pl.ds(0, W)], wt_ccw.at[slot], wt_sems.at[1]).wait()

    def out_wait():
        pltpu.make_async_copy(res_cw, out_ref.at[:, pl.ds(0, W)],
                              out_sems.at[0]).wait()
        pltpu.make_async_copy(res_ccw, out_ref.at[:, pl.ds(0, W)],
                              out_sems.at[1]).wait()

    def pair_body(p, amax_val):
        slot = lax.rem(p, 2)
        col_cw = pl.ds(p * W, W)
        col_ccw = pl.ds((p + N_PAIRS) * W, W)
        wcw = wt_cw.at[slot]
        wccw = wt_ccw.at[slot]

        for s in range(3):
            if s == 0:
                src_cw, src_ccw = pre_cw, pre_ccw
            else:
                sbuf_cw[...] = buf_cw[s - 1] + mmb_cw[...]
                sbuf_ccw[...] = buf_ccw[s - 1] + mmb_ccw[...]
                src_cw, src_ccw = sbuf_cw, sbuf_ccw
            rdma_cw = pltpu.make_async_remote_copy(
                src_ref=src_cw, dst_ref=buf_cw.at[s],
                send_sem=send_cw.at[s], recv_sem=recv_cw.at[s],
                device_id=(right,), device_id_type=pl.DeviceIdType.MESH,
            )
            rdma_ccw = pltpu.make_async_remote_copy(
                src_ref=src_ccw, dst_ref=buf_ccw.at[s],
                send_sem=send_ccw.at[s], recv_sem=recv_ccw.at[s],
                device_id=(left,), device_id_type=pl.DeviceIdType.MESH,
            )
            rdma_cw.start()
            rdma_ccw.start()
            if s == 0:
                @pl.when(p + 1 < N_PAIRS)
                def _():
                    wt_fetch(p + 1, lax.rem(p + 1, 2))
            c_cw = lax.rem(d - 2 - s + 2 * N_DEV, N_DEV)
            c_ccw = lax.rem(d + 2 + s, N_DEV)
            mmb_cw[...] = _mm(x_ref, c_cw, wcw[...])
            mmb_ccw[...] = _mm(x_ref, c_ccw, wccw[...])
            if s == 1:
                @pl.when(p + 1 < N_PAIRS)
                def _():
                    nslot = lax.rem(p + 1, 2)
                    wt_wait(nslot)
                    pre_cw[...] = _mm(x_ref, lax.rem(d - 1 + N_DEV, N_DEV),
                                      wt_cw[nslot])
            if s == 2:
                @pl.when(p + 1 < N_PAIRS)
                def _():
                    nslot = lax.rem(p + 1, 2)
                    pre_ccw[...] = _mm(x_ref, lax.rem(d + 1, N_DEV),
                                       wt_ccw[nslot])
            rdma_cw.wait()
            rdma_ccw.wait()

        @pl.when(p > 0)
        def _():
            out_wait()
        res_cw[...] = buf_cw[2] + mmb_cw[...]
        res_ccw[...] = buf_ccw[2] + mmb_ccw[...]
        amax_val = jnp.maximum(amax_val, jnp.max(jnp.abs(res_cw[...])))
        amax_val = jnp.maximum(amax_val, jnp.max(jnp.abs(res_ccw[...])))
        pltpu.make_async_copy(res_cw, out_ref.at[:, col_cw],
                              out_sems.at[0]).start()
        pltpu.make_async_copy(res_ccw, out_ref.at[:, col_ccw],
                              out_sems.at[1]).start()
        return amax_val

    wt_fetch(0, 0)
    wt_wait(0)
    pre_cw[...] = _mm(x_ref, lax.rem(d - 1 + N_DEV, N_DEV), wt_cw[0])
    pre_ccw[...] = _mm(x_ref, lax.rem(d + 1, N_DEV), wt_ccw[0])

    barrier = pltpu.get_barrier_semaphore()
    for nbr in (left, right):
        pl.semaphore_signal(
            barrier, inc=1,
            device_id=(nbr,), device_id_type=pl.DeviceIdType.MESH,
        )
    pl.semaphore_wait(barrier, 2)

    amax_val = lax.fori_loop(0, N_PAIRS, pair_body,
                             jnp.zeros((), jnp.float32))

    axbuf_ref[3, :, :] = jnp.full((8, 128), amax_val, jnp.float32)
    sends = []
    for off in (1, 2, 3):
        tgt = lax.rem(d + off, N_DEV)
        rdma = pltpu.make_async_remote_copy(
            src_ref=axbuf_ref.at[3],
            dst_ref=axbuf_ref.at[off - 1],
            send_sem=ax_send.at[off - 1],
            recv_sem=ax_recv.at[off - 1],
            device_id=(tgt,),
            device_id_type=pl.DeviceIdType.MESH,
        )
        rdma.start()
        sends.append(rdma)
    out_wait()
    for rdma in sends:
        rdma.wait_send()
    for rdma in sends:
        rdma.wait_recv()
    amax_ref[...] = jnp.max(axbuf_ref[...]).reshape(1, 1)


def _gemm_rs_amax(x, w_mat):
    return pl.pallas_call(
        _body,
        out_shape=[
            jax.ShapeDtypeStruct((M_PER, N_COLS), jnp.float32),
            jax.ShapeDtypeStruct((1, 1), jnp.float32),
        ],
        in_specs=[
            pl.BlockSpec(memory_space=pltpu.VMEM),
            pl.BlockSpec(memory_space=pl.ANY),
        ],
        out_specs=[
            pl.BlockSpec(memory_space=pl.ANY),
            pl.BlockSpec(memory_space=pltpu.VMEM),
        ],
        scratch_shapes=[
            pltpu.VMEM((3, M_PER, W), jnp.float32),
            pltpu.VMEM((3, M_PER, W), jnp.float32),
            pltpu.VMEM((2, K_PER, W), jnp.float32),
            pltpu.VMEM((2, K_PER, W), jnp.float32),
            pltpu.VMEM((M_PER, W), jnp.float32),
            pltpu.VMEM((M_PER, W), jnp.float32),
            pltpu.VMEM((M_PER, W), jnp.float32),
            pltpu.VMEM((M_PER, W), jnp.float32),
            pltpu.VMEM((M_PER, W), jnp.float32),
            pltpu.VMEM((M_PER, W), jnp.float32),
            pltpu.VMEM((M_PER, W), jnp.float32),
            pltpu.VMEM((M_PER, W), jnp.float32),
            pltpu.VMEM((N_DEV, 8, 128), jnp.float32),
            pltpu.SemaphoreType.DMA((2,)),
            pltpu.SemaphoreType.DMA((2,)),
            pltpu.SemaphoreType.DMA((3,)),
            pltpu.SemaphoreType.DMA((3,)),
            pltpu.SemaphoreType.DMA((3,)),
            pltpu.SemaphoreType.DMA((3,)),
            pltpu.SemaphoreType.DMA((3,)),
            pltpu.SemaphoreType.DMA((3,)),
        ],
        compiler_params=pltpu.CompilerParams(
            collective_id=0,
            vmem_limit_bytes=int(63.9 * 1024 * 1024)),
    )(x, w_mat)


def kernel(x, w_mat):
    acc, amax = _gemm_rs_amax(x, w_mat)
    scale = amax[0, 0] / 448.0
    q = (acc / scale).astype(jnp.float8_e4m3fn)
    q = lax.optimization_barrier(q)
    return q.astype(jnp.float32) * scale
